# baseline (speedup 1.0000x reference)
import functools

import jax
import jax.numpy as jnp
from jax import lax
from jax.experimental import pallas as pl
from jax.experimental.pallas import tpu as pltpu
from jax.experimental.pallas import tpu_sc as plsc

E = 200000
N = 10000
K = 20
EDGE_FEATS = 128
TIME_DIM = 100
HIDDEN = 100
HP = 128
OUT_DIM = 100

BE = 2000
NB = 400
_INV_SQRT2 = 0.7071067811865475


def _gelu(v):
    return 0.5 * v * (1.0 + jax.lax.erf(v * _INV_SQRT2))


def _enc_body(ef_ref, et_ref, tw_ref, wef_ref, wt_ref, b_ref, o_ref):
    et_enc = jnp.cos(et_ref[...] * tw_ref[...])
    acc = jnp.dot(ef_ref[...], wef_ref[...], preferred_element_type=jnp.float32)
    acc = acc + jnp.dot(et_enc, wt_ref[...], preferred_element_type=jnp.float32)
    o_ref[...] = acc + b_ref[...]


def _encode(ef, et, time_w, W_lin, b_lin):
    wef = jnp.zeros((EDGE_FEATS, HP), jnp.float32).at[:, :HIDDEN].set(W_lin[:, :EDGE_FEATS].T)
    wt = jnp.zeros((TIME_DIM, HP), jnp.float32).at[:, :HIDDEN].set(W_lin[:, EDGE_FEATS:].T)
    b2 = jnp.zeros((1, HP), jnp.float32).at[:, :HIDDEN].set(b_lin.reshape(1, HIDDEN))
    tw = time_w.reshape(1, TIME_DIM)
    et2 = et.reshape(E, 1)
    return pl.pallas_call(
        _enc_body,
        grid=(E // BE,),
        in_specs=[
            pl.BlockSpec((BE, EDGE_FEATS), lambda i: (i, 0)),
            pl.BlockSpec((BE, 1), lambda i: (i, 0)),
            pl.BlockSpec((1, TIME_DIM), lambda i: (0, 0)),
            pl.BlockSpec((EDGE_FEATS, HP), lambda i: (0, 0)),
            pl.BlockSpec((TIME_DIM, HP), lambda i: (0, 0)),
            pl.BlockSpec((1, HP), lambda i: (0, 0)),
        ],
        out_specs=pl.BlockSpec((BE, HP), lambda i: (i, 0)),
        out_shape=jax.ShapeDtypeStruct((E, HP), jnp.float32),
    )(ef, et2, tw, wef, wt, b2)


SHARD = 12512
E_PAD = 16 * SHARD
NV = SHARD // 16
NPASS = 16
ROWS_FULL = 6336
ROWS_LAST = 4960
SP_ROWS = ROWS_FULL + 64
ZCH = SP_ROWS // 16
CHUNK = 128


def _sc_scatter(nid_p, idx_p, x, zeros_z):
    mesh = plsc.VectorSubcoreMesh(core_axis_name="c", subcore_axis_name="s",
                                  num_cores=2, num_subcores=16)

    @functools.partial(
        pl.kernel,
        out_type=jax.ShapeDtypeStruct((K * N, HP), jnp.float32),
        mesh=mesh,
        scratch_types=[
            pltpu.VMEM((SHARD,), jnp.int32),
            pltpu.VMEM((SHARD,), jnp.int32),
            pltpu.VMEM((SHARD,), jnp.int32),
            pltpu.VMEM((SHARD + CHUNK,), jnp.int32),
            pltpu.VMEM((SHARD + CHUNK,), jnp.int32),
            pltpu.VMEM((1, CHUNK), jnp.int32),
            pltpu.VMEM((CHUNK, HP), jnp.float32),
            pltpu.VMEM_SHARED((SP_ROWS, HP), jnp.float32),
            pltpu.SemaphoreType.DMA,
        ],
        compiler_params=pltpu.CompilerParams(needs_layout_passes=False),
    )
    def scat(nid_hbm, idx_hbm, x_hbm, z_hbm, out_hbm,
             nid_v, idx_v, pos_v, midx_v, meid_v, istg_v, rowbuf_v,
             acc_sh, sem):
        c = lax.axis_index("c")
        s = lax.axis_index("s")
        base = s * SHARD
        pltpu.sync_copy(nid_hbm.at[pl.ds(base, SHARD)], nid_v)
        pltpu.sync_copy(idx_hbm.at[pl.ds(base, SHARD)], idx_v)

        def posbody(v, _):
            sl = pl.ds(v * 16, 16)
            pos_v[sl] = idx_v[sl] * N + nid_v[sl]
            return 0
        lax.fori_loop(0, NV, posbody, 0)

        lanes = lax.iota(jnp.int32, 16)

        for p in range(NPASS):
            rows_p = ROWS_LAST if p == NPASS - 1 else ROWS_FULL
            lo = p * (2 * ROWS_FULL) + c * rows_p
            pltpu.sync_copy(z_hbm, acc_sh.at[pl.ds(s * ZCH, ZCH)])
            plsc.subcore_barrier()

            def fbody(v, cnt, lo=lo, rows_p=rows_p):
                pv = pos_v[pl.ds(v * 16, 16)]
                lidx = pv - lo
                m = (lidx >= 0) & (lidx < rows_p)
                r = cnt + plsc.cumsum(m.astype(jnp.int32)) - 1
                plsc.store_scatter(midx_v, [r], lidx, mask=m)
                plsc.store_scatter(meid_v, [r], lanes + (base + v * 16), mask=m)
                return cnt + jnp.sum(m.astype(jnp.int32))
            cnt = lax.fori_loop(0, NV, fbody, 0)

            trash = jnp.full((16,), ROWS_FULL + s, jnp.int32)
            eidpad = jnp.full((16,), base, jnp.int32)
            for q in range(CHUNK // 16):
                plsc.store_scatter(midx_v, [cnt + q * 16 + lanes], trash)
                plsc.store_scatter(meid_v, [cnt + q * 16 + lanes], eidpad)
            nch = (cnt + CHUNK - 1) // CHUNK

            def cbody(j, _):
                for q in range(CHUNK // 16):
                    istg_v[0, pl.ds(q * 16, 16)] = midx_v[pl.ds(j * CHUNK + q * 16, 16)]
                pltpu.async_copy(
                    x_hbm.at[meid_v.at[pl.ds(j * CHUNK, CHUNK)]], rowbuf_v, sem
                ).wait()
                pltpu.sync_copy(rowbuf_v, acc_sh.at[istg_v.at[0]], add=True)
                return 0
            lax.fori_loop(0, nch, cbody, 0)
            plsc.subcore_barrier()

            if p < NPASS - 1:
                st = s * 392
                pltpu.sync_copy(acc_sh.at[pl.ds(st, 392)],
                                out_hbm.at[pl.ds(lo + st, 392)])
                @pl.when(s == 0)
                def _(lo=lo):
                    pltpu.sync_copy(acc_sh.at[pl.ds(6272, 64)],
                                    out_hbm.at[pl.ds(lo + 6272, 64)])
            else:
                st = s * 304
                pltpu.sync_copy(acc_sh.at[pl.ds(st, 304)],
                                out_hbm.at[pl.ds(lo + st, 304)])
                @pl.when(s == 0)
                def _(lo=lo):
                    pltpu.sync_copy(acc_sh.at[pl.ds(4864, 96)],
                                    out_hbm.at[pl.ds(lo + 4864, 96)])
            plsc.subcore_barrier()

    return scat(nid_p, idx_p, x, zeros_z)


BT = 51200


def _token_body(x_ref, cg_ref, cb_ref, wc1_ref, bc1_ref, wc2_ref, bc2_ref, o_ref):
    eps = 1e-5
    x = x_ref[...]
    mu = jnp.mean(x, axis=0, keepdims=True)
    var = jnp.mean((x - mu) ** 2, axis=0, keepdims=True)
    z = (x - mu) * jax.lax.rsqrt(var + eps) * cg_ref[...] + cb_ref[...]
    h = jnp.dot(wc1_ref[...], z, preferred_element_type=jnp.float32) + bc1_ref[...]
    y = jnp.dot(wc2_ref[...], _gelu(h), preferred_element_type=jnp.float32) + bc2_ref[...]
    o_ref[...] = x + y


def _token_mix(split2, cg, cb, Wc1, bc1, Wc2, bc2):
    Kd2 = K // 2
    return pl.pallas_call(
        _token_body,
        grid=(N * HP // BT,),
        in_specs=[
            pl.BlockSpec((K, BT), lambda i: (0, i)),
            pl.BlockSpec((K, 1), lambda i: (0, 0)),
            pl.BlockSpec((K, 1), lambda i: (0, 0)),
            pl.BlockSpec((Kd2, K), lambda i: (0, 0)),
            pl.BlockSpec((Kd2, 1), lambda i: (0, 0)),
            pl.BlockSpec((K, Kd2), lambda i: (0, 0)),
            pl.BlockSpec((K, 1), lambda i: (0, 0)),
        ],
        out_specs=pl.BlockSpec((K, BT), lambda i: (0, i)),
        out_shape=jax.ShapeDtypeStruct((K, N * HP), jnp.float32),
    )(split2, cg.reshape(K, 1), cb.reshape(K, 1), Wc1, bc1.reshape(Kd2, 1),
      Wc2, bc2.reshape(K, 1))


def _mixer_body(x_ref, rg_ref, rb_ref, wr1_ref, br1_ref, wr2_ref, br2_ref,
                ng_ref, nb_ref, wo_ref, bo_ref, o_ref):
    eps = 1e-5
    x = x_ref[...]
    hm = (lax.broadcasted_iota(jnp.int32, (NB, HP), 1) < HIDDEN).astype(jnp.float32)
    inv_h = 1.0 / HIDDEN

    pooled = None
    for k in range(K):
        xk = x[k]
        mu2 = jnp.sum(xk * hm, axis=-1, keepdims=True) * inv_h
        d2 = xk - mu2
        var2 = jnp.sum(d2 * d2 * hm, axis=-1, keepdims=True) * inv_h
        z2 = d2 * jax.lax.rsqrt(var2 + eps) * rg_ref[...] + rb_ref[...]
        h1 = jnp.dot(z2, wr1_ref[...], preferred_element_type=jnp.float32) + br1_ref[...]
        h2 = jnp.dot(_gelu(h1), wr2_ref[...], preferred_element_type=jnp.float32) + br2_ref[...]
        xk = xk + h2
        mu3 = jnp.sum(xk * hm, axis=-1, keepdims=True) * inv_h
        d3 = xk - mu3
        var3 = jnp.sum(d3 * d3 * hm, axis=-1, keepdims=True) * inv_h
        z3 = d3 * jax.lax.rsqrt(var3 + eps) * ng_ref[...] + nb_ref[...]
        pooled = z3 if pooled is None else pooled + z3
    pooled = pooled * (1.0 / K)
    o_ref[...] = jnp.dot(pooled, wo_ref[...], preferred_element_type=jnp.float32) + bo_ref[...]


def _pad_cols(a, width=HP):
    out = jnp.zeros((a.shape[0], width), a.dtype)
    return out.at[:, :a.shape[1]].set(a)


def _mix(split, rg, rb, Wr1, br1, Wr2, br2, ng, nb_, W_out, b_out):
    wr1p = jnp.zeros((HP, 4 * HIDDEN), jnp.float32).at[:HIDDEN, :].set(Wr1.T)
    wr2p = _pad_cols(Wr2.T)
    wop = jnp.zeros((HP, OUT_DIM), jnp.float32).at[:HIDDEN, :].set(W_out.T)
    rgp = _pad_cols(rg.reshape(1, HIDDEN))
    rbp = _pad_cols(rb.reshape(1, HIDDEN))
    ngp = _pad_cols(ng.reshape(1, HIDDEN))
    nbp = _pad_cols(nb_.reshape(1, HIDDEN))
    br2p = _pad_cols(br2.reshape(1, HIDDEN))
    return pl.pallas_call(
        _mixer_body,
        grid=(N // NB,),
        in_specs=[
            pl.BlockSpec((K, NB, HP), lambda i: (0, i, 0)),
            pl.BlockSpec((1, HP), lambda i: (0, 0)),
            pl.BlockSpec((1, HP), lambda i: (0, 0)),
            pl.BlockSpec((HP, 4 * HIDDEN), lambda i: (0, 0)),
            pl.BlockSpec((1, 4 * HIDDEN), lambda i: (0, 0)),
            pl.BlockSpec((4 * HIDDEN, HP), lambda i: (0, 0)),
            pl.BlockSpec((1, HP), lambda i: (0, 0)),
            pl.BlockSpec((1, HP), lambda i: (0, 0)),
            pl.BlockSpec((1, HP), lambda i: (0, 0)),
            pl.BlockSpec((HP, OUT_DIM), lambda i: (0, 0)),
            pl.BlockSpec((1, OUT_DIM), lambda i: (0, 0)),
        ],
        out_specs=pl.BlockSpec((NB, OUT_DIM), lambda i: (i, 0)),
        out_shape=jax.ShapeDtypeStruct((N, OUT_DIM), jnp.float32),
    )(split, rgp, rbp, wr1p, br1.reshape(1, 4 * HIDDEN), wr2p, br2p,
      ngp, nbp, wop, b_out.reshape(1, OUT_DIM))


def kernel(ef, et, nid, idx, time_w, W_lin, b_lin, cg, cb, Wc1, bc1, Wc2, bc2,
           rg, rb, Wr1, br1, Wr2, br2, ng, nb, W_out, b_out):
    x = _encode(ef, et, time_w, W_lin, b_lin)
    pad = E_PAD - E
    nid_p = jnp.concatenate([nid, jnp.full((pad,), -1, jnp.int32)])
    idx_p = jnp.concatenate([idx, jnp.zeros((pad,), jnp.int32)])
    zeros_z = jnp.zeros((ZCH, HP), jnp.float32)
    split = _sc_scatter(nid_p, idx_p, x, zeros_z)
    tm = _token_mix(split.reshape(K, N * HP), cg, cb, Wc1, bc1, Wc2, bc2)
    return _mix(tm.reshape(K, N, HP), rg, rb, Wr1, br1,
                Wr2, br2, ng, nb, W_out, b_out)


# speedup vs baseline: 3.1950x; 3.1950x over previous
import functools

import jax
import jax.numpy as jnp
from jax import lax
from jax.experimental import pallas as pl
from jax.experimental.pallas import tpu as pltpu
from jax.experimental.pallas import tpu_sc as plsc

E = 200000
N = 10000
K = 20
EDGE_FEATS = 128
TIME_DIM = 100
HIDDEN = 100
HP = 128
OUT_DIM = 100

BE = 2000
NB = 400
_INV_SQRT2 = 0.7071067811865475


def _gelu(v):
    return 0.5 * v * (1.0 + jax.lax.erf(v * _INV_SQRT2))


def _enc_body(ef_ref, et_ref, tw_ref, wef_ref, wt_ref, b_ref, o_ref):
    et_enc = jnp.cos(et_ref[...] * tw_ref[...])
    acc = jnp.dot(ef_ref[...], wef_ref[...], preferred_element_type=jnp.float32)
    acc = acc + jnp.dot(et_enc, wt_ref[...], preferred_element_type=jnp.float32)
    o_ref[...] = acc + b_ref[...]


def _encode(ef, et, time_w, W_lin, b_lin):
    wef = jnp.zeros((EDGE_FEATS, HP), jnp.float32).at[:, :HIDDEN].set(W_lin[:, :EDGE_FEATS].T)
    wt = jnp.zeros((TIME_DIM, HP), jnp.float32).at[:, :HIDDEN].set(W_lin[:, EDGE_FEATS:].T)
    b2 = jnp.zeros((1, HP), jnp.float32).at[:, :HIDDEN].set(b_lin.reshape(1, HIDDEN))
    tw = time_w.reshape(1, TIME_DIM)
    et2 = et.reshape(E, 1)
    return pl.pallas_call(
        _enc_body,
        grid=(E // BE,),
        in_specs=[
            pl.BlockSpec((BE, EDGE_FEATS), lambda i: (i, 0)),
            pl.BlockSpec((BE, 1), lambda i: (i, 0)),
            pl.BlockSpec((1, TIME_DIM), lambda i: (0, 0)),
            pl.BlockSpec((EDGE_FEATS, HP), lambda i: (0, 0)),
            pl.BlockSpec((TIME_DIM, HP), lambda i: (0, 0)),
            pl.BlockSpec((1, HP), lambda i: (0, 0)),
        ],
        out_specs=pl.BlockSpec((BE, HP), lambda i: (i, 0)),
        out_shape=jax.ShapeDtypeStruct((E, HP), jnp.float32),
    )(ef, et2, tw, wef, wt, b2)


SHARD = 12512
E_PAD = 16 * SHARD
NV = SHARD // 16
NPASS = 16
ROWS_FULL = 6336
ROWS_LAST = 4960
SP_ROWS = ROWS_FULL + 64
ZCH = SP_ROWS // 16
CHUNK = 128


def _sc_scatter(nid_p, idx_p, x, zeros_z):
    mesh = plsc.VectorSubcoreMesh(core_axis_name="c", subcore_axis_name="s",
                                  num_cores=2, num_subcores=16)

    @functools.partial(
        pl.kernel,
        out_type=jax.ShapeDtypeStruct((K * N, HP), jnp.float32),
        mesh=mesh,
        scratch_types=[
            pltpu.VMEM((SHARD,), jnp.int32),
            pltpu.VMEM((SHARD + CHUNK,), jnp.int32),
            pltpu.VMEM((SHARD + CHUNK,), jnp.int32),
            pltpu.VMEM((1, CHUNK), jnp.int32),
            pltpu.VMEM((1, CHUNK), jnp.int32),
            pltpu.VMEM((CHUNK, HP), jnp.float32),
            pltpu.VMEM((CHUNK, HP), jnp.float32),
            pltpu.VMEM_SHARED((SP_ROWS, HP), jnp.float32),
            pltpu.SemaphoreType.DMA,
            pltpu.SemaphoreType.DMA,
        ],
        compiler_params=pltpu.CompilerParams(needs_layout_passes=False),
    )
    def scat(nid_hbm, idx_hbm, x_hbm, z_hbm, out_hbm,
             pos_v, midx_v, meid_v, istg0_v, istg1_v,
             rowbuf0_v, rowbuf1_v, acc_sh, sem0, sem1):
        c = lax.axis_index("c")
        s = lax.axis_index("s")
        base = s * SHARD
        pltpu.sync_copy(nid_hbm.at[pl.ds(base, SHARD)], pos_v)
        pltpu.sync_copy(idx_hbm.at[pl.ds(base, SHARD)], midx_v.at[pl.ds(0, SHARD)])

        def posbody(v, _):
            sl = pl.ds(v * 16, 16)
            pos_v[sl] = midx_v[sl] * N + pos_v[sl]
            return 0
        lax.fori_loop(0, NV, posbody, 0)

        lanes = lax.iota(jnp.int32, 16)

        for p in range(NPASS):
            rows_p = ROWS_LAST if p == NPASS - 1 else ROWS_FULL
            lo = p * (2 * ROWS_FULL) + c * rows_p
            pltpu.sync_copy(z_hbm, acc_sh.at[pl.ds(s * ZCH, ZCH)])
            plsc.subcore_barrier()

            def fbody(v, cnt, lo=lo, rows_p=rows_p):
                pv = pos_v[pl.ds(v * 16, 16)]
                lidx = pv - lo
                m = (lidx >= 0) & (lidx < rows_p)
                r = cnt + plsc.cumsum(m.astype(jnp.int32)) - 1
                plsc.store_scatter(midx_v, [r], lidx, mask=m)
                plsc.store_scatter(meid_v, [r], lanes + (base + v * 16), mask=m)
                return cnt + jnp.sum(m.astype(jnp.int32))
            cnt = lax.fori_loop(0, NV, fbody, 0)

            trash = jnp.full((16,), ROWS_FULL + s, jnp.int32)
            eidpad = jnp.full((16,), base, jnp.int32)
            for q in range(CHUNK // 16):
                plsc.store_scatter(midx_v, [cnt + q * 16 + lanes], trash)
                plsc.store_scatter(meid_v, [cnt + q * 16 + lanes], eidpad)
            nch = (cnt + CHUNK - 1) // CHUNK

            def _stage(istg, j):
                for q in range(CHUNK // 16):
                    istg[0, pl.ds(q * 16, 16)] = midx_v[pl.ds(j * CHUNK + q * 16, 16)]

            @pl.when(nch > 0)
            def _():
                pltpu.async_copy(
                    x_hbm.at[meid_v.at[pl.ds(0, CHUNK)]], rowbuf0_v, sem0)

            def pair(g, _):
                j0 = 2 * g
                j1 = j0 + 1
                pltpu.make_async_copy(
                    x_hbm.at[meid_v.at[pl.ds(0, CHUNK)]], rowbuf0_v, sem0).wait()
                @pl.when(j1 < nch)
                def _():
                    pltpu.async_copy(
                        x_hbm.at[meid_v.at[pl.ds(j1 * CHUNK, CHUNK)]],
                        rowbuf1_v, sem1)
                _stage(istg0_v, j0)
                pltpu.sync_copy(rowbuf0_v, acc_sh.at[istg0_v.at[0]], add=True)
                @pl.when(j1 < nch)
                def _():
                    pltpu.make_async_copy(
                        x_hbm.at[meid_v.at[pl.ds(0, CHUNK)]], rowbuf1_v, sem1).wait()
                    @pl.when(j1 + 1 < nch)
                    def _():
                        pltpu.async_copy(
                            x_hbm.at[meid_v.at[pl.ds((j1 + 1) * CHUNK, CHUNK)]],
                            rowbuf0_v, sem0)
                    _stage(istg1_v, j1)
                    pltpu.sync_copy(rowbuf1_v, acc_sh.at[istg1_v.at[0]], add=True)
                return 0
            lax.fori_loop(0, (nch + 1) // 2, pair, 0)
            plsc.subcore_barrier()

            if p < NPASS - 1:
                st = s * 392
                pltpu.sync_copy(acc_sh.at[pl.ds(st, 392)],
                                out_hbm.at[pl.ds(lo + st, 392)])
                @pl.when(s == 0)
                def _(lo=lo):
                    pltpu.sync_copy(acc_sh.at[pl.ds(6272, 64)],
                                    out_hbm.at[pl.ds(lo + 6272, 64)])
            else:
                st = s * 304
                pltpu.sync_copy(acc_sh.at[pl.ds(st, 304)],
                                out_hbm.at[pl.ds(lo + st, 304)])
                @pl.when(s == 0)
                def _(lo=lo):
                    pltpu.sync_copy(acc_sh.at[pl.ds(4864, 96)],
                                    out_hbm.at[pl.ds(lo + 4864, 96)])
            plsc.subcore_barrier()

    return scat(nid_p, idx_p, x, zeros_z)


def _mixer_body(x_ref, cg_ref, cb_ref, wc1_ref, bc1_ref, wc2_ref, bc2_ref,
                rg_ref, rb_ref, wr1_ref, br1_ref, wr2_ref, br2_ref,
                ng_ref, nb_ref, wo_ref, bo_ref, o_ref):
    eps = 1e-5
    x = x_ref[...]
    Kd2 = K // 2
    hm = (lax.broadcasted_iota(jnp.int32, (NB, HP), 1) < HIDDEN).astype(jnp.float32)
    inv_h = 1.0 / HIDDEN

    mu = jnp.mean(x, axis=0, keepdims=True)
    var = jnp.mean((x - mu) ** 2, axis=0, keepdims=True)
    inv = jax.lax.rsqrt(var + eps)
    zs = [(x[k] - mu[0]) * inv[0] * cg_ref[0, k] + cb_ref[0, k] for k in range(K)]
    hs = []
    for j in range(Kd2):
        acc = zs[0] * wc1_ref[j, 0]
        for k in range(1, K):
            acc = acc + zs[k] * wc1_ref[j, k]
        hs.append(_gelu(acc + bc1_ref[0, j]))
    xs = []
    for k in range(K):
        acc = hs[0] * wc2_ref[k, 0]
        for j in range(1, Kd2):
            acc = acc + hs[j] * wc2_ref[k, j]
        xs.append(x[k] + acc + bc2_ref[0, k])

    pooled = None
    for k in range(K):
        xk = xs[k]
        mu2 = jnp.sum(xk * hm, axis=-1, keepdims=True) * inv_h
        d2 = xk - mu2
        var2 = jnp.sum(d2 * d2 * hm, axis=-1, keepdims=True) * inv_h
        z2 = d2 * jax.lax.rsqrt(var2 + eps) * rg_ref[...] + rb_ref[...]
        h1 = jnp.dot(z2, wr1_ref[...], preferred_element_type=jnp.float32) + br1_ref[...]
        h2 = jnp.dot(_gelu(h1), wr2_ref[...], preferred_element_type=jnp.float32) + br2_ref[...]
        xk = xk + h2
        mu3 = jnp.sum(xk * hm, axis=-1, keepdims=True) * inv_h
        d3 = xk - mu3
        var3 = jnp.sum(d3 * d3 * hm, axis=-1, keepdims=True) * inv_h
        z3 = d3 * jax.lax.rsqrt(var3 + eps) * ng_ref[...] + nb_ref[...]
        pooled = z3 if pooled is None else pooled + z3
    pooled = pooled * (1.0 / K)
    o_ref[...] = jnp.dot(pooled, wo_ref[...], preferred_element_type=jnp.float32) + bo_ref[...]


def _pad_cols(a, width=HP):
    out = jnp.zeros((a.shape[0], width), a.dtype)
    return out.at[:, :a.shape[1]].set(a)


def _mix(split, cg, cb, Wc1, bc1, Wc2, bc2, rg, rb, Wr1, br1, Wr2, br2,
         ng, nb_, W_out, b_out):
    Kd2 = K // 2
    wr1p = jnp.zeros((HP, 4 * HIDDEN), jnp.float32).at[:HIDDEN, :].set(Wr1.T)
    wr2p = _pad_cols(Wr2.T)
    wop = jnp.zeros((HP, OUT_DIM), jnp.float32).at[:HIDDEN, :].set(W_out.T)
    rgp = _pad_cols(rg.reshape(1, HIDDEN))
    rbp = _pad_cols(rb.reshape(1, HIDDEN))
    ngp = _pad_cols(ng.reshape(1, HIDDEN))
    nbp = _pad_cols(nb_.reshape(1, HIDDEN))
    br2p = _pad_cols(br2.reshape(1, HIDDEN))
    return pl.pallas_call(
        _mixer_body,
        grid=(N // NB,),
        in_specs=[pl.BlockSpec((K, NB, HP), lambda i: (0, i, 0))]
        + [pl.BlockSpec(memory_space=pltpu.SMEM)] * 6
        + [
            pl.BlockSpec((1, HP), lambda i: (0, 0)),
            pl.BlockSpec((1, HP), lambda i: (0, 0)),
            pl.BlockSpec((HP, 4 * HIDDEN), lambda i: (0, 0)),
            pl.BlockSpec((1, 4 * HIDDEN), lambda i: (0, 0)),
            pl.BlockSpec((4 * HIDDEN, HP), lambda i: (0, 0)),
            pl.BlockSpec((1, HP), lambda i: (0, 0)),
            pl.BlockSpec((1, HP), lambda i: (0, 0)),
            pl.BlockSpec((1, HP), lambda i: (0, 0)),
            pl.BlockSpec((HP, OUT_DIM), lambda i: (0, 0)),
            pl.BlockSpec((1, OUT_DIM), lambda i: (0, 0)),
        ],
        out_specs=pl.BlockSpec((NB, OUT_DIM), lambda i: (i, 0)),
        out_shape=jax.ShapeDtypeStruct((N, OUT_DIM), jnp.float32),
    )(split, cg.reshape(1, K), cb.reshape(1, K), Wc1, bc1.reshape(1, Kd2),
      Wc2, bc2.reshape(1, K),
      rgp, rbp, wr1p, br1.reshape(1, 4 * HIDDEN), wr2p, br2p,
      ngp, nbp, wop, b_out.reshape(1, OUT_DIM))


def kernel(ef, et, nid, idx, time_w, W_lin, b_lin, cg, cb, Wc1, bc1, Wc2, bc2,
           rg, rb, Wr1, br1, Wr2, br2, ng, nb, W_out, b_out):
    x = _encode(ef, et, time_w, W_lin, b_lin)
    pad = E_PAD - E
    nid_p = jnp.concatenate([nid, jnp.full((pad,), -1, jnp.int32)])
    idx_p = jnp.concatenate([idx, jnp.zeros((pad,), jnp.int32)])
    zeros_z = jnp.zeros((ZCH, HP), jnp.float32)
    split = _sc_scatter(nid_p, idx_p, x, zeros_z)
    return _mix(split.reshape(K, N, HP), cg, cb, Wc1, bc1, Wc2, bc2,
                rg, rb, Wr1, br1, Wr2, br2, ng, nb, W_out, b_out)


# speedup vs baseline: 3.2289x; 1.0106x over previous
import functools

import jax
import jax.numpy as jnp
from jax import lax
from jax.experimental import pallas as pl
from jax.experimental.pallas import tpu as pltpu
from jax.experimental.pallas import tpu_sc as plsc

E = 200000
N = 10000
K = 20
EDGE_FEATS = 128
TIME_DIM = 100
HIDDEN = 100
HP = 128
OUT_DIM = 100

BE = 2000
NB = 400
_INV_SQRT2 = 0.7071067811865475


def _gelu(v):
    return 0.5 * v * (1.0 + jax.lax.erf(v * _INV_SQRT2))


def _enc_body(ef_ref, et_ref, tw_ref, wef_ref, wt_ref, b_ref, o_ref):
    et_enc = jnp.cos(et_ref[...] * tw_ref[...])
    acc = jnp.dot(ef_ref[...], wef_ref[...], preferred_element_type=jnp.float32)
    acc = acc + jnp.dot(et_enc, wt_ref[...], preferred_element_type=jnp.float32)
    o_ref[...] = acc + b_ref[...]


def _encode(ef, et, time_w, W_lin, b_lin):
    wef = jnp.zeros((EDGE_FEATS, HP), jnp.float32).at[:, :HIDDEN].set(W_lin[:, :EDGE_FEATS].T)
    wt = jnp.zeros((TIME_DIM, HP), jnp.float32).at[:, :HIDDEN].set(W_lin[:, EDGE_FEATS:].T)
    b2 = jnp.zeros((1, HP), jnp.float32).at[:, :HIDDEN].set(b_lin.reshape(1, HIDDEN))
    tw = time_w.reshape(1, TIME_DIM)
    et2 = et.reshape(E, 1)
    return pl.pallas_call(
        _enc_body,
        grid=(E // BE,),
        in_specs=[
            pl.BlockSpec((BE, EDGE_FEATS), lambda i: (i, 0)),
            pl.BlockSpec((BE, 1), lambda i: (i, 0)),
            pl.BlockSpec((1, TIME_DIM), lambda i: (0, 0)),
            pl.BlockSpec((EDGE_FEATS, HP), lambda i: (0, 0)),
            pl.BlockSpec((TIME_DIM, HP), lambda i: (0, 0)),
            pl.BlockSpec((1, HP), lambda i: (0, 0)),
        ],
        out_specs=pl.BlockSpec((BE, HP), lambda i: (i, 0)),
        out_shape=jax.ShapeDtypeStruct((E, HP), jnp.float32),
    )(ef, et2, tw, wef, wt, b2)


SHARD = 12512
E_PAD = 16 * SHARD
NV = SHARD // 16
NPASS = 16
ROWS_FULL = 6336
ROWS_LAST = 4960
SP_ROWS = ROWS_FULL + 64
ZCH = SP_ROWS // 16
CHUNK = 128


def _sc_scatter(nid_p, idx_p, x, zeros_z):
    mesh = plsc.VectorSubcoreMesh(core_axis_name="c", subcore_axis_name="s",
                                  num_cores=2, num_subcores=16)

    @functools.partial(
        pl.kernel,
        out_type=jax.ShapeDtypeStruct((K * N, HP), jnp.float32),
        mesh=mesh,
        scratch_types=[
            pltpu.VMEM((SHARD,), jnp.int32),
            pltpu.VMEM((SHARD + CHUNK,), jnp.int32),
            pltpu.VMEM((SHARD + CHUNK,), jnp.int32),
            pltpu.VMEM((1, CHUNK), jnp.int32),
            pltpu.VMEM((1, CHUNK), jnp.int32),
            pltpu.VMEM((CHUNK, HP), jnp.float32),
            pltpu.VMEM((CHUNK, HP), jnp.float32),
            pltpu.VMEM_SHARED((SP_ROWS, HP), jnp.float32),
            pltpu.SemaphoreType.DMA,
            pltpu.SemaphoreType.DMA,
        ],
        compiler_params=pltpu.CompilerParams(needs_layout_passes=False),
    )
    def scat(nid_hbm, idx_hbm, x_hbm, z_hbm, out_hbm,
             pos_v, midx_v, meid_v, istg0_v, istg1_v,
             rowbuf0_v, rowbuf1_v, acc_sh, sem0, sem1):
        c = lax.axis_index("c")
        s = lax.axis_index("s")
        base = s * SHARD
        pltpu.sync_copy(nid_hbm.at[pl.ds(base, SHARD)], pos_v)
        pltpu.sync_copy(idx_hbm.at[pl.ds(base, SHARD)], midx_v.at[pl.ds(0, SHARD)])

        def posbody(v, _):
            sl = pl.ds(v * 16, 16)
            pos_v[sl] = midx_v[sl] * N + pos_v[sl]
            return 0
        lax.fori_loop(0, NV, posbody, 0)

        lanes = lax.iota(jnp.int32, 16)

        for p in range(NPASS):
            rows_p = ROWS_LAST if p == NPASS - 1 else ROWS_FULL
            lo = p * (2 * ROWS_FULL) + c * rows_p
            pltpu.sync_copy(z_hbm, acc_sh.at[pl.ds(s * ZCH, ZCH)])
            plsc.subcore_barrier()

            def fbody(v, cnt, lo=lo, rows_p=rows_p):
                pv = pos_v[pl.ds(v * 16, 16)]
                lidx = pv - lo
                m = (lidx >= 0) & (lidx < rows_p)
                plsc.store_compressed(midx_v.at[pl.ds(cnt, 16)], lidx, mask=m)
                plsc.store_compressed(meid_v.at[pl.ds(cnt, 16)],
                                      lanes + (base + v * 16), mask=m)
                return cnt + plsc.all_reduce_population_count(m)[0]
            cnt = lax.fori_loop(0, NV, fbody, 0)

            trash = jnp.full((16,), ROWS_FULL + s, jnp.int32)
            eidpad = jnp.full((16,), base, jnp.int32)
            for q in range(CHUNK // 16):
                plsc.store_scatter(midx_v, [cnt + q * 16 + lanes], trash)
                plsc.store_scatter(meid_v, [cnt + q * 16 + lanes], eidpad)
            nch = (cnt + CHUNK - 1) // CHUNK

            def _stage(istg, j):
                for q in range(CHUNK // 16):
                    istg[0, pl.ds(q * 16, 16)] = midx_v[pl.ds(j * CHUNK + q * 16, 16)]

            @pl.when(nch > 0)
            def _():
                pltpu.async_copy(
                    x_hbm.at[meid_v.at[pl.ds(0, CHUNK)]], rowbuf0_v, sem0)

            def pair(g, _):
                j0 = 2 * g
                j1 = j0 + 1
                pltpu.make_async_copy(
                    x_hbm.at[meid_v.at[pl.ds(0, CHUNK)]], rowbuf0_v, sem0).wait()
                @pl.when(j1 < nch)
                def _():
                    pltpu.async_copy(
                        x_hbm.at[meid_v.at[pl.ds(j1 * CHUNK, CHUNK)]],
                        rowbuf1_v, sem1)
                _stage(istg0_v, j0)
                pltpu.sync_copy(rowbuf0_v, acc_sh.at[istg0_v.at[0]], add=True)
                @pl.when(j1 < nch)
                def _():
                    pltpu.make_async_copy(
                        x_hbm.at[meid_v.at[pl.ds(0, CHUNK)]], rowbuf1_v, sem1).wait()
                    @pl.when(j1 + 1 < nch)
                    def _():
                        pltpu.async_copy(
                            x_hbm.at[meid_v.at[pl.ds((j1 + 1) * CHUNK, CHUNK)]],
                            rowbuf0_v, sem0)
                    _stage(istg1_v, j1)
                    pltpu.sync_copy(rowbuf1_v, acc_sh.at[istg1_v.at[0]], add=True)
                return 0
            lax.fori_loop(0, (nch + 1) // 2, pair, 0)
            plsc.subcore_barrier()

            if p < NPASS - 1:
                st = s * 392
                pltpu.sync_copy(acc_sh.at[pl.ds(st, 392)],
                                out_hbm.at[pl.ds(lo + st, 392)])
                @pl.when(s == 0)
                def _(lo=lo):
                    pltpu.sync_copy(acc_sh.at[pl.ds(6272, 64)],
                                    out_hbm.at[pl.ds(lo + 6272, 64)])
            else:
                st = s * 304
                pltpu.sync_copy(acc_sh.at[pl.ds(st, 304)],
                                out_hbm.at[pl.ds(lo + st, 304)])
                @pl.when(s == 0)
                def _(lo=lo):
                    pltpu.sync_copy(acc_sh.at[pl.ds(4864, 96)],
                                    out_hbm.at[pl.ds(lo + 4864, 96)])
            plsc.subcore_barrier()

    return scat(nid_p, idx_p, x, zeros_z)


def _mixer_body(x_ref, cg_ref, cb_ref, wc1_ref, bc1_ref, wc2_ref, bc2_ref,
                rg_ref, rb_ref, wr1_ref, br1_ref, wr2_ref, br2_ref,
                ng_ref, nb_ref, wo_ref, bo_ref, o_ref):
    eps = 1e-5
    x = x_ref[...]
    Kd2 = K // 2
    hm = (lax.broadcasted_iota(jnp.int32, (NB, HP), 1) < HIDDEN).astype(jnp.float32)
    inv_h = 1.0 / HIDDEN

    mu = jnp.mean(x, axis=0, keepdims=True)
    var = jnp.mean((x - mu) ** 2, axis=0, keepdims=True)
    inv = jax.lax.rsqrt(var + eps)
    zs = [(x[k] - mu[0]) * inv[0] * cg_ref[0, k] + cb_ref[0, k] for k in range(K)]
    hs = []
    for j in range(Kd2):
        acc = zs[0] * wc1_ref[j, 0]
        for k in range(1, K):
            acc = acc + zs[k] * wc1_ref[j, k]
        hs.append(_gelu(acc + bc1_ref[0, j]))
    xs = []
    for k in range(K):
        acc = hs[0] * wc2_ref[k, 0]
        for j in range(1, Kd2):
            acc = acc + hs[j] * wc2_ref[k, j]
        xs.append(x[k] + acc + bc2_ref[0, k])

    pooled = None
    for k in range(K):
        xk = xs[k]
        mu2 = jnp.sum(xk * hm, axis=-1, keepdims=True) * inv_h
        d2 = xk - mu2
        var2 = jnp.sum(d2 * d2 * hm, axis=-1, keepdims=True) * inv_h
        z2 = d2 * jax.lax.rsqrt(var2 + eps) * rg_ref[...] + rb_ref[...]
        h1 = jnp.dot(z2, wr1_ref[...], preferred_element_type=jnp.float32) + br1_ref[...]
        h2 = jnp.dot(_gelu(h1), wr2_ref[...], preferred_element_type=jnp.float32) + br2_ref[...]
        xk = xk + h2
        mu3 = jnp.sum(xk * hm, axis=-1, keepdims=True) * inv_h
        d3 = xk - mu3
        var3 = jnp.sum(d3 * d3 * hm, axis=-1, keepdims=True) * inv_h
        z3 = d3 * jax.lax.rsqrt(var3 + eps) * ng_ref[...] + nb_ref[...]
        pooled = z3 if pooled is None else pooled + z3
    pooled = pooled * (1.0 / K)
    o_ref[...] = jnp.dot(pooled, wo_ref[...], preferred_element_type=jnp.float32) + bo_ref[...]


def _pad_cols(a, width=HP):
    out = jnp.zeros((a.shape[0], width), a.dtype)
    return out.at[:, :a.shape[1]].set(a)


def _mix(split, cg, cb, Wc1, bc1, Wc2, bc2, rg, rb, Wr1, br1, Wr2, br2,
         ng, nb_, W_out, b_out):
    Kd2 = K // 2
    wr1p = jnp.zeros((HP, 4 * HIDDEN), jnp.float32).at[:HIDDEN, :].set(Wr1.T)
    wr2p = _pad_cols(Wr2.T)
    wop = jnp.zeros((HP, OUT_DIM), jnp.float32).at[:HIDDEN, :].set(W_out.T)
    rgp = _pad_cols(rg.reshape(1, HIDDEN))
    rbp = _pad_cols(rb.reshape(1, HIDDEN))
    ngp = _pad_cols(ng.reshape(1, HIDDEN))
    nbp = _pad_cols(nb_.reshape(1, HIDDEN))
    br2p = _pad_cols(br2.reshape(1, HIDDEN))
    return pl.pallas_call(
        _mixer_body,
        grid=(N // NB,),
        in_specs=[pl.BlockSpec((K, NB, HP), lambda i: (0, i, 0))]
        + [pl.BlockSpec(memory_space=pltpu.SMEM)] * 6
        + [
            pl.BlockSpec((1, HP), lambda i: (0, 0)),
            pl.BlockSpec((1, HP), lambda i: (0, 0)),
            pl.BlockSpec((HP, 4 * HIDDEN), lambda i: (0, 0)),
            pl.BlockSpec((1, 4 * HIDDEN), lambda i: (0, 0)),
            pl.BlockSpec((4 * HIDDEN, HP), lambda i: (0, 0)),
            pl.BlockSpec((1, HP), lambda i: (0, 0)),
            pl.BlockSpec((1, HP), lambda i: (0, 0)),
            pl.BlockSpec((1, HP), lambda i: (0, 0)),
            pl.BlockSpec((HP, OUT_DIM), lambda i: (0, 0)),
            pl.BlockSpec((1, OUT_DIM), lambda i: (0, 0)),
        ],
        out_specs=pl.BlockSpec((NB, OUT_DIM), lambda i: (i, 0)),
        out_shape=jax.ShapeDtypeStruct((N, OUT_DIM), jnp.float32),
    )(split, cg.reshape(1, K), cb.reshape(1, K), Wc1, bc1.reshape(1, Kd2),
      Wc2, bc2.reshape(1, K),
      rgp, rbp, wr1p, br1.reshape(1, 4 * HIDDEN), wr2p, br2p,
      ngp, nbp, wop, b_out.reshape(1, OUT_DIM))


def kernel(ef, et, nid, idx, time_w, W_lin, b_lin, cg, cb, Wc1, bc1, Wc2, bc2,
           rg, rb, Wr1, br1, Wr2, br2, ng, nb, W_out, b_out):
    x = _encode(ef, et, time_w, W_lin, b_lin)
    pad = E_PAD - E
    nid_p = jnp.concatenate([nid, jnp.full((pad,), -1, jnp.int32)])
    idx_p = jnp.concatenate([idx, jnp.zeros((pad,), jnp.int32)])
    zeros_z = jnp.zeros((ZCH, HP), jnp.float32)
    split = _sc_scatter(nid_p, idx_p, x, zeros_z)
    return _mix(split.reshape(K, N, HP), cg, cb, Wc1, bc1, Wc2, bc2,
                rg, rb, Wr1, br1, Wr2, br2, ng, nb, W_out, b_out)
